# baseline (speedup 1.0000x reference)
import functools

import jax
import jax.numpy as jnp
from jax import lax
from jax.experimental import pallas as pl
from jax.experimental.pallas import tpu as pltpu
from jax.experimental.pallas import tpu_sc as plsc

H = 128
T = 200
CHUNK = 8
L = 16
NC, NS = 2, 16
NW = NC * NS
GCHUNKS = ((0, 0, 128), (1, 128, 72))
TPAD = 256
HV = H // L


def _pool_body(
    ids_hbm, table_hbm, out_hbm, idx_v, rows_v, acc_v, sem0, sem1, sem2, sem3
):
    rpb = TPAD // 128
    bpw = ids_hbm.shape[0] // rpb // NW
    wid = lax.axis_index("s") * NC + lax.axis_index("c")
    base = wid * bpw
    half = bpw // 2
    sems = (sem0, sem1, sem2, sem3)

    def stage_idx(k):
        pltpu.sync_copy(
            ids_hbm.at[pl.ds((base + k * half) * rpb, half * rpb)], idx_v
        )

    def fire(bl, slot):
        for row, off, ln in GCHUNKS:
            pltpu.async_copy(
                table_hbm.at[idx_v.at[bl * rpb + row, pl.ds(0, ln)]],
                rows_v.at[slot, pl.ds(off, ln)],
                sems[slot],
            )

    def drain(slot):
        for _, off, ln in GCHUNKS:
            pltpu.make_async_copy(
                table_hbm.at[pl.ds(0, ln)],
                rows_v.at[slot, pl.ds(off, ln)],
                sems[slot],
            ).wait()

    def accum(b, slot):
        def body(t, accs):
            return tuple(
                accs[h] + rows_v[slot, t, pl.ds(h * L, L)] for h in range(HV)
            )
        accs = tuple(jnp.zeros((L,), jnp.float32) for _ in range(HV))
        accs = plsc.parallel_loop(0, T, 1, unroll=4, carry=accs)(body)
        for h in range(HV):
            acc_v[b % 16, pl.ds(h * L, L)] = accs[h] * (1.0 / T)

        @pl.when(b % 16 == 15)
        def _():
            pltpu.sync_copy(acc_v, out_hbm.at[pl.ds(base + b - 15, 16)])

    def run_block(rb):
        fire(0, 0)
        fire(1, 1)
        fire(2, 2)

        def step(i, _):
            bl0 = 4 * i
            b0 = rb + bl0
            fire(bl0 + 3, 3)
            drain(0)
            accum(b0, 0)
            fire(bl0 + 4, 0)
            drain(1)
            accum(b0 + 1, 1)
            fire(bl0 + 5, 1)
            drain(2)
            accum(b0 + 2, 2)
            fire(bl0 + 6, 2)
            drain(3)
            accum(b0 + 3, 3)
            return 0

        lax.fori_loop(0, half // 4 - 1, step, 0)
        fire(half - 1, 3)
        drain(0)
        accum(rb + half - 4, 0)
        drain(1)
        accum(rb + half - 3, 1)
        drain(2)
        accum(rb + half - 2, 2)
        drain(3)
        accum(rb + half - 1, 3)

    stage_idx(0)
    run_block(0)
    stage_idx(1)
    run_block(half)


def _pooled(ids_sc, emb_table):
    batch = ids_sc.shape[0] * 128 // TPAD
    bpw = batch // NW
    mesh = plsc.VectorSubcoreMesh(
        core_axis_name="c", subcore_axis_name="s", num_cores=NC, num_subcores=NS
    )
    f = functools.partial(
        pl.kernel,
        mesh=mesh,
        compiler_params=pltpu.CompilerParams(use_tc_tiling_on_sc=False),
        out_type=jax.ShapeDtypeStruct((batch, H), jnp.float32),
        scratch_types=[
            pltpu.VMEM((bpw * TPAD // 128 // 2, 128), jnp.int32),
            pltpu.VMEM((4, T, H), jnp.float32),
            pltpu.VMEM((16, H), jnp.float32),
            pltpu.SemaphoreType.DMA,
            pltpu.SemaphoreType.DMA,
            pltpu.SemaphoreType.DMA,
            pltpu.SemaphoreType.DMA,
        ],
    )(_pool_body)
    return f(ids_sc, emb_table)


def _proj_body(x_ref, wt_ref, b_ref, o_ref):
    x = x_ref[...]
    for c in range(CHUNK):
        o_ref[:, c, :] = (
            jnp.dot(x, wt_ref[:, c, :], preferred_element_type=jnp.float32)
            + b_ref[c, :]
        )


def _proj(pooled, wt, bias):
    bm = 512
    batch = pooled.shape[0]
    return pl.pallas_call(
        _proj_body,
        grid=(batch // bm,),
        in_specs=[
            pl.BlockSpec((bm, H), lambda i: (i, 0)),
            pl.BlockSpec((H, CHUNK, H), lambda i: (0, 0, 0)),
            pl.BlockSpec((CHUNK, H), lambda i: (0, 0)),
        ],
        out_specs=pl.BlockSpec((bm, CHUNK, H), lambda i: (i, 0, 0)),
        out_shape=jax.ShapeDtypeStruct((batch, CHUNK, H), jnp.float32),
    )(pooled, wt, bias)


def kernel(thought_ids, emb_table, W, b):
    batch = thought_ids.shape[0]
    wt = W.T.reshape(H, CHUNK, H)
    bias = b.reshape(CHUNK, H)
    ids_sc = jnp.pad(thought_ids, ((0, 0), (0, TPAD - T))).reshape(-1, 128)
    pooled = _pooled(ids_sc, emb_table)
    return _proj(pooled, wt, bias)
